# baseline (speedup 1.0000x reference)
import functools

import jax
import jax.numpy as jnp
from jax.experimental import pallas as pl
from jax.experimental.pallas import tpu as pltpu

BATCH = 8
SRCLEN = 64
VOCAB = 8000
DMODEL = 512
BEAM = 4
MAXLEN = 12
PAD, UNK, BOS, EOS = 0, 1, 2, 3

_NEG = -1e30
_NCHUNK = 4


def _fused_kernel(tok_smem, len_ref, semb_hbm, encw_ref, temb_hbm, dw_ref,
                  ow_hbm, out_ref,
                  esrc_ref, e_ref, tokv_ref, toks_ref, ow_ref,
                  gsem, sem, dsem):
    d = DMODEL
    BS = BATCH * SRCLEN

    def _issue(i, _):
        t = tok_smem[i, 0]
        pltpu.make_async_copy(semb_hbm.at[pl.ds(t, 1), :],
                              esrc_ref.at[pl.ds(i, 1), :], gsem).start()
        return 0
    jax.lax.fori_loop(0, BS, _issue, 0, unroll=False)

    cp_eos = pltpu.make_async_copy(temb_hbm.at[EOS:EOS + 1, :],
                                   e_ref.at[0:1, :], dsem.at[0])
    cp_eos.start()
    DC = DMODEL // _NCHUNK
    cp_ow = []
    for c in range(_NCHUNK):
        cp = pltpu.make_async_copy(ow_hbm.at[pl.ds(c * DC, DC), :],
                                   ow_ref.at[pl.ds(c * DC, DC), :],
                                   dsem.at[1 + c])
        cp.start()
        cp_ow.append(cp)

    def _wait(i, _):
        pltpu.make_async_copy(semb_hbm.at[pl.ds(0, 1), :],
                              esrc_ref.at[pl.ds(0, 1), :], gsem).wait()
        return 0
    jax.lax.fori_loop(0, BS, _wait, 0, unroll=False)

    enc = jnp.tanh(jax.lax.dot_general(
        esrc_ref[:], encw_ref[:], (((1,), (0,)), ((), ())),
        preferred_element_type=jnp.float32))
    lens = len_ref[:]

    colidx = jax.lax.broadcasted_iota(jnp.int32, (BATCH, BS), 1)
    rowidx = jax.lax.broadcasted_iota(jnp.int32, (BATCH, BS), 0)
    att_ok = (colidx // SRCLEN == rowidx) & (colidx % SRCLEN < lens)

    viota = jax.lax.broadcasted_iota(jnp.int32, (BATCH, VOCAB), 1)
    special = viota < 3
    inv_sqrt_d = jnp.float32(1.0) / jnp.sqrt(jnp.float32(d))

    out_ref[:] = jnp.zeros((BATCH, 16), jnp.int32)
    out_ref[:, 0:1] = jnp.full((BATCH, 1), EOS, jnp.int32)

    last = jnp.full((BATCH, 1), EOS, jnp.int32)
    done = jnp.zeros((1, 1), jnp.float32)

    for p in range(MAXLEN - 1):
        if p == 0:
            cp_eos.wait()
            e = jnp.broadcast_to(e_ref[0:1, :], (BATCH, d))
        else:
            for b in range(BATCH):
                pltpu.make_async_copy(temb_hbm.at[pl.ds(toks_ref[b, 0], 1), :],
                                      e_ref.at[pl.ds(b, 1), :], gsem).start()
            for b in range(BATCH):
                pltpu.make_async_copy(temb_hbm.at[pl.ds(0, 1), :],
                                      e_ref.at[pl.ds(0, 1), :], gsem).wait()
            e = e_ref[:]
        h = jax.lax.dot_general(e, dw_ref[:], (((1,), (0,)), ((), ())),
                                preferred_element_type=jnp.float32)
        att = jax.lax.dot_general(h, enc, (((1,), (1,)), ((), ())),
                                  preferred_element_type=jnp.float32)
        att = jnp.where(att_ok, att * inv_sqrt_d, _NEG)
        att = att - jnp.max(att, axis=1, keepdims=True)
        att = jnp.exp(att)
        att = att / jnp.sum(att, axis=1, keepdims=True)
        ctx = jax.lax.dot_general(att, enc, (((1,), (0,)), ((), ())),
                                  preferred_element_type=jnp.float32)
        if p == 0:
            for cp in cp_ow:
                cp.wait()
        logits = jax.lax.dot_general(h + ctx, ow_ref[:], (((1,), (0,)), ((), ())),
                                     preferred_element_type=jnp.float32)
        logits = jnp.where(special, _NEG, logits)
        m = jnp.max(logits, axis=1, keepdims=True)
        amax = jnp.min(jnp.where(logits == m, viota, VOCAB),
                       axis=1, keepdims=True).astype(jnp.int32)
        if p >= 1:
            nxt = jnp.where(last == EOS, EOS, amax)
        else:
            nxt = amax
        out_ref[:, p + 1:p + 2] = jnp.where(done > 0.5, 0, nxt)
        all_eos = jnp.min((nxt == EOS).astype(jnp.float32), axis=0, keepdims=True)
        done = jnp.maximum(done, all_eos)
        last = nxt
        if p < MAXLEN - 2:
            tokv_ref[:] = jnp.broadcast_to(nxt, (BATCH, 128))
            cp = pltpu.make_async_copy(tokv_ref, toks_ref, sem)
            cp.start()
            cp.wait()

    final = jnp.where(done > 0.5, 0, EOS)
    out_ref[:, MAXLEN:MAXLEN + 1] = jnp.broadcast_to(final, (BATCH, 1)).astype(jnp.int32)


@jax.jit
def kernel(src_tokens, src_lengths, tgt_tokens, src_embed, enc_W, tgt_embed, dec_W, out_W):
    del tgt_tokens
    tok = src_tokens.astype(jnp.int32).reshape(BATCH * SRCLEN, 1)
    lens = src_lengths.astype(jnp.int32).reshape(BATCH, 1)
    gen = pl.pallas_call(
        _fused_kernel,
        out_shape=jax.ShapeDtypeStruct((BATCH, 16), jnp.int32),
        in_specs=[
            pl.BlockSpec(memory_space=pltpu.SMEM),
            pl.BlockSpec(memory_space=pltpu.VMEM),
            pl.BlockSpec(memory_space=pltpu.HBM),
            pl.BlockSpec(memory_space=pltpu.VMEM),
            pl.BlockSpec(memory_space=pltpu.HBM),
            pl.BlockSpec(memory_space=pltpu.VMEM),
            pl.BlockSpec(memory_space=pltpu.HBM),
        ],
        scratch_shapes=[
            pltpu.VMEM((BATCH * SRCLEN, DMODEL), jnp.float32),
            pltpu.VMEM((BATCH, DMODEL), jnp.float32),
            pltpu.VMEM((BATCH, 128), jnp.int32),
            pltpu.SMEM((BATCH, 128), jnp.int32),
            pltpu.VMEM((DMODEL, VOCAB), jnp.float32),
            pltpu.SemaphoreType.DMA,
            pltpu.SemaphoreType.DMA,
            pltpu.SemaphoreType.DMA((1 + _NCHUNK,)),
        ],
    )(tok, lens, src_embed, enc_W, tgt_embed, dec_W, out_W)
    return gen[:, 1:MAXLEN + 2]


# speedup vs baseline: 1.1288x; 1.1288x over previous
import functools

import jax
import jax.numpy as jnp
from jax.experimental import pallas as pl
from jax.experimental.pallas import tpu as pltpu

BATCH = 8
SRCLEN = 64
VOCAB = 8000
DMODEL = 512
BEAM = 4
MAXLEN = 12
PAD, UNK, BOS, EOS = 0, 1, 2, 3

_NEG = -1e30
_NCHUNK = 4


def _fused_kernel(tok_smem, len_ref, semb_hbm, encw_ref, temb_hbm, dw_ref,
                  ow_hbm, out_ref,
                  esrc_ref, e_ref, tokv_ref, toks_ref, temb_ref, ow_ref,
                  gsem, sem, dsem):
    d = DMODEL
    BS = BATCH * SRCLEN

    cp_eos = pltpu.make_async_copy(temb_hbm.at[EOS:EOS + 1, :],
                                   e_ref.at[0:1, :], dsem.at[0])
    cp_eos.start()
    VC = VOCAB // _NCHUNK
    DC = DMODEL // _NCHUNK
    cp_ow = []
    cp_temb = []
    for c in range(_NCHUNK):
        cp = pltpu.make_async_copy(ow_hbm.at[pl.ds(c * DC, DC), :],
                                   ow_ref.at[pl.ds(c * DC, DC), :],
                                   dsem.at[1 + c])
        cp.start()
        cp_ow.append(cp)

    def _issue(i, _):
        t = tok_smem[i, 0]
        pltpu.make_async_copy(semb_hbm.at[pl.ds(t, 1), :],
                              esrc_ref.at[pl.ds(i, 1), :], gsem).start()
        return 0
    jax.lax.fori_loop(0, BS, _issue, 0, unroll=False)

    for c in range(_NCHUNK):
        cp = pltpu.make_async_copy(temb_hbm.at[pl.ds(c * VC, VC), :],
                                   temb_ref.at[pl.ds(c * VC, VC), :],
                                   dsem.at[1 + _NCHUNK + c])
        cp.start()
        cp_temb.append(cp)

    def _wait(i, _):
        pltpu.make_async_copy(semb_hbm.at[pl.ds(0, 1), :],
                              esrc_ref.at[pl.ds(0, 1), :], gsem).wait()
        return 0
    jax.lax.fori_loop(0, BS, _wait, 0, unroll=False)

    enc = jnp.tanh(jax.lax.dot_general(
        esrc_ref[:], encw_ref[:], (((1,), (0,)), ((), ())),
        preferred_element_type=jnp.float32))
    lens = len_ref[:]

    colidx = jax.lax.broadcasted_iota(jnp.int32, (BATCH, BS), 1)
    rowidx = jax.lax.broadcasted_iota(jnp.int32, (BATCH, BS), 0)
    att_ok = (colidx // SRCLEN == rowidx) & (colidx % SRCLEN < lens)

    viota = jax.lax.broadcasted_iota(jnp.int32, (BATCH, VOCAB), 1)
    special = viota < 3
    inv_sqrt_d = jnp.float32(1.0) / jnp.sqrt(jnp.float32(d))

    out_ref[:] = jnp.zeros((BATCH, 16), jnp.int32)
    out_ref[:, 0:1] = jnp.full((BATCH, 1), EOS, jnp.int32)

    last = jnp.full((BATCH, 1), EOS, jnp.int32)
    done = jnp.zeros((1, 1), jnp.float32)

    for p in range(MAXLEN - 1):
        if p == 0:
            cp_eos.wait()
            e = jnp.broadcast_to(e_ref[0:1, :], (BATCH, d))
        else:
            if p == 1:
                for cp in cp_temb:
                    cp.wait()
            for b in range(BATCH):
                e_ref[b:b + 1, :] = temb_ref[pl.ds(toks_ref[b, 0], 1), :]
            e = e_ref[:]
        h = jax.lax.dot_general(e, dw_ref[:], (((1,), (0,)), ((), ())),
                                preferred_element_type=jnp.float32)
        att = jax.lax.dot_general(h, enc, (((1,), (1,)), ((), ())),
                                  preferred_element_type=jnp.float32)
        att = jnp.where(att_ok, att * inv_sqrt_d, _NEG)
        att = att - jnp.max(att, axis=1, keepdims=True)
        att = jnp.exp(att)
        att = att / jnp.sum(att, axis=1, keepdims=True)
        ctx = jax.lax.dot_general(att, enc, (((1,), (0,)), ((), ())),
                                  preferred_element_type=jnp.float32)
        if p == 0:
            for cp in cp_ow:
                cp.wait()
        logits = jax.lax.dot_general(h + ctx, ow_ref[:], (((1,), (0,)), ((), ())),
                                     preferred_element_type=jnp.float32)
        logits = jnp.where(special, _NEG, logits)
        m = jnp.max(logits, axis=1, keepdims=True)
        amax = jnp.min(jnp.where(logits == m, viota, VOCAB),
                       axis=1, keepdims=True).astype(jnp.int32)
        if p >= 1:
            nxt = jnp.where(last == EOS, EOS, amax)
        else:
            nxt = amax
        out_ref[:, p + 1:p + 2] = jnp.where(done > 0.5, 0, nxt)
        all_eos = jnp.min((nxt == EOS).astype(jnp.float32), axis=0, keepdims=True)
        done = jnp.maximum(done, all_eos)
        last = nxt
        if p < MAXLEN - 2:
            tokv_ref[:] = jnp.broadcast_to(nxt, (BATCH, 128))
            cp = pltpu.make_async_copy(tokv_ref, toks_ref, sem)
            cp.start()
            cp.wait()

    final = jnp.where(done > 0.5, 0, EOS)
    out_ref[:, MAXLEN:MAXLEN + 1] = jnp.broadcast_to(final, (BATCH, 1)).astype(jnp.int32)


@jax.jit
def kernel(src_tokens, src_lengths, tgt_tokens, src_embed, enc_W, tgt_embed, dec_W, out_W):
    del tgt_tokens
    tok = src_tokens.astype(jnp.int32).reshape(BATCH * SRCLEN, 1)
    lens = src_lengths.astype(jnp.int32).reshape(BATCH, 1)
    gen = pl.pallas_call(
        _fused_kernel,
        out_shape=jax.ShapeDtypeStruct((BATCH, 16), jnp.int32),
        in_specs=[
            pl.BlockSpec(memory_space=pltpu.SMEM),
            pl.BlockSpec(memory_space=pltpu.VMEM),
            pl.BlockSpec(memory_space=pltpu.HBM),
            pl.BlockSpec(memory_space=pltpu.VMEM),
            pl.BlockSpec(memory_space=pltpu.HBM),
            pl.BlockSpec(memory_space=pltpu.VMEM),
            pl.BlockSpec(memory_space=pltpu.HBM),
        ],
        scratch_shapes=[
            pltpu.VMEM((BATCH * SRCLEN, DMODEL), jnp.float32),
            pltpu.VMEM((BATCH, DMODEL), jnp.float32),
            pltpu.VMEM((BATCH, 128), jnp.int32),
            pltpu.SMEM((BATCH, 128), jnp.int32),
            pltpu.VMEM((VOCAB, DMODEL), jnp.float32),
            pltpu.VMEM((DMODEL, VOCAB), jnp.float32),
            pltpu.SemaphoreType.DMA,
            pltpu.SemaphoreType.DMA,
            pltpu.SemaphoreType.DMA((1 + 2 * _NCHUNK,)),
        ],
    )(tok, lens, src_embed, enc_W, tgt_embed, dec_W, out_W)
    return gen[:, 1:MAXLEN + 2]
